# Initial kernel scaffold; baseline (speedup 1.0000x reference)
import jax
import jax.numpy as jnp
from jax.experimental import pallas as pl


def kernel(inputs, theta):
    raise NotImplementedError("write your pallas kernel here")


# speedup vs baseline: 33.9573x; 33.9573x over previous
import jax
import jax.numpy as jnp
import numpy as np
from jax.experimental import pallas as pl
from jax.experimental.pallas import tpu as pltpu

_I32_MIN = np.int32(-2147483648)
_I32_LOW = np.int32(2147483647)


def _select_body(theta_ref, x_ref, o_ref):
    R, L = x_ref.shape
    x = x_ref[...]
    th = theta_ref[0, 0]
    act = 1.0 / (1.0 + jnp.exp(-th))
    k = jnp.floor(L * (1.0 - act)).astype(jnp.int32)

    bits = jax.lax.bitcast_convert_type(x, jnp.int32)
    key = jnp.where(bits < 0, bits ^ _I32_LOW, bits)

    def step(i, p):
        b = jnp.left_shift(jnp.int32(1), 31 - i)
        cand_u = p | b
        cand_s = cand_u ^ _I32_MIN
        cnt = jnp.sum((key >= cand_s).astype(jnp.int32), axis=1, keepdims=True)
        return jnp.where(cnt >= k, cand_u, p)

    p = jax.lax.fori_loop(0, 32, step, jnp.zeros((R, 1), jnp.int32))
    q_s = p ^ _I32_MIN
    mid_s = jnp.min(q_s)
    mid_bits = jnp.where(mid_s < 0, mid_s ^ _I32_LOW, mid_s)
    mid = jax.lax.bitcast_convert_type(mid_bits, jnp.float32)

    z = x - mid
    ez = jnp.exp(-jnp.abs(z))
    t = 1.0 / (1.0 + ez)
    o_ref[...] = jnp.where(z >= 0, t, 1.0 - t)


def kernel(inputs, theta):
    R, L = inputs.shape
    theta2d = jnp.reshape(theta, (1, 1))
    return pl.pallas_call(
        _select_body,
        out_shape=jax.ShapeDtypeStruct((R, L), jnp.float32),
        in_specs=[
            pl.BlockSpec(memory_space=pltpu.SMEM),
            pl.BlockSpec(memory_space=pltpu.VMEM),
        ],
        out_specs=pl.BlockSpec(memory_space=pltpu.VMEM),
    )(theta2d, inputs)
